# baseline (speedup 1.0000x reference)
import functools

import jax
import jax.numpy as jnp
from jax import lax
from jax.experimental import pallas as pl
from jax.experimental.pallas import tpu as pltpu
from jax.experimental.pallas import tpu_sc as plsc

BATCH = 4
SEQ = 8192
D = 1024

NC = 2
NS = 16
NW = NC * NS

ROWS_PER_W = SEQ // NW
R = 16
CHUNKS = ROWS_PER_W // R
CHUNK_ELEMS = R * D
VECS = CHUNK_ELEMS // 16
STEPS = CHUNKS * BATCH

_mesh = plsc.VectorSubcoreMesh(core_axis_name="c", subcore_axis_name="s")


@functools.partial(
    pl.kernel,
    mesh=_mesh,
    out_type=jax.ShapeDtypeStruct((BATCH * SEQ * D,), jnp.float32),
    scratch_types=[
        pltpu.VMEM((CHUNK_ELEMS,), jnp.float32),
        pltpu.VMEM((CHUNK_ELEMS,), jnp.float32),
        pltpu.VMEM((CHUNK_ELEMS,), jnp.float32),
        pltpu.VMEM((CHUNK_ELEMS,), jnp.float32),
        pltpu.VMEM((CHUNK_ELEMS,), jnp.float32),
        pltpu.VMEM((CHUNK_ELEMS,), jnp.float32),
        pltpu.SemaphoreType.DMA,
        pltpu.SemaphoreType.DMA,
        pltpu.SemaphoreType.DMA,
        pltpu.SemaphoreType.DMA,
        pltpu.SemaphoreType.DMA,
        pltpu.SemaphoreType.DMA,
        pltpu.SemaphoreType.DMA,
        pltpu.SemaphoreType.DMA,
        pltpu.SemaphoreType.DMA,
        pltpu.SemaphoreType.DMA,
    ],
)
def _sc_add(x_hbm, pos_hbm, out_hbm,
            xb0, xb1, xb2, xb3, pb0, pb1,
            in0, in1, in2, in3, o0, o1, o2, o3, ps0, ps1):
    wid = lax.axis_index("s") * NC + lax.axis_index("c")
    row0 = wid * ROWS_PER_W
    pos_base = row0 * D

    xbufs = (xb0, xb1, xb2, xb3)
    pbufs = (pb0, pb1)
    in_sems = (in0, in1, in2, in3)
    out_sems = (o0, o1, o2, o3)
    pos_sems = (ps0, ps1)

    def x_off(step):
        c, b = step // BATCH, step % BATCH
        return b * (SEQ * D) + pos_base + c * CHUNK_ELEMS

    def start_x(step):
        p = step % 4
        return pltpu.async_copy(
            x_hbm.at[pl.ds(x_off(step), CHUNK_ELEMS)], xbufs[p], in_sems[p])

    def start_pos(c):
        p = c % 2
        return pltpu.async_copy(
            pos_hbm.at[pl.ds(pos_base + c * CHUNK_ELEMS, CHUNK_ELEMS)],
            pbufs[p], pos_sems[p])

    def start_out(step):
        p = step % 4
        return pltpu.async_copy(
            xbufs[p], out_hbm.at[pl.ds(x_off(step), CHUNK_ELEMS)], out_sems[p])

    in_flight = {}
    pos_flight = {}
    out_flight = {}

    pos_flight[0] = start_pos(0)
    in_flight[0] = start_x(0)
    in_flight[1] = start_x(1)

    for s in range(STEPS):
        p = s % 4
        c = s // BATCH
        nxt = s + 2
        if nxt < STEPS:
            if nxt - 4 >= 0:
                out_flight[nxt - 4].wait()
            in_flight[nxt] = start_x(nxt)
        if s % BATCH == 3 and c + 1 < CHUNKS:
            pos_flight[c + 1] = start_pos(c + 1)

        in_flight[s].wait()
        if s % BATCH == 0:
            pos_flight[c].wait()

        xb = xbufs[p]
        pb = pbufs[c % 2]

        def add8(k, carry, xb=xb, pb=pb):
            base = k * 128
            for j in range(8):
                sl = pl.ds(base + j * 16, 16)
                xb[sl] = xb[sl] + pb[sl]
            return carry

        lax.fori_loop(0, VECS // 8, add8, 0)

        out_flight[s] = start_out(s)

    for s in range(STEPS - 4, STEPS):
        out_flight[s].wait()


def kernel(x, pos_table):
    out = _sc_add(x.reshape(-1), pos_table.reshape(-1))
    return out.reshape(x.shape)
